# baseline (speedup 1.0000x reference)
import functools

import jax
import jax.numpy as jnp
from jax import lax
from jax.experimental import pallas as pl
from jax.experimental.pallas import tpu as pltpu
from jax.experimental.pallas import tpu_sc as plsc

VOCAB = 100000
D = 128
B = 1024
H = 200

NUM_CORES = 2
NUM_SUBCORES = 16
NW = NUM_CORES * NUM_SUBCORES
BPW = B // NW
NCHUNK = 2
CHL = H // NCHUNK
LANES = 16
NVREG = D // LANES


NBUF = 8
RUNROLL = 4


def _sc_body(x_hbm, table_hbm, out_hbm, idx_v,
             b0, b1, b2, b3, b4, b5, b6, b7, out_v,
             s0, s1, s2, s3, s4, s5, s6, s7):
    c = lax.axis_index("c")
    s = lax.axis_index("s")
    wid = s * NUM_CORES + c
    base = wid * BPW

    pltpu.sync_copy(x_hbm.at[pl.ds(base, BPW)], idx_v)

    bufs = (b0, b1, b2, b3, b4, b5, b6, b7)
    sems = (s0, s1, s2, s3, s4, s5, s6, s7)
    SPG = NBUF // NCHUNK

    def issue(b, si, h):
        pltpu.async_copy(table_hbm.at[idx_v.at[si, h]], bufs[b], sems[b])

    def wait(b):
        pltpu.make_async_copy(
            table_hbm.at[idx_v.at[0, 0]], bufs[b], sems[b]).wait()

    def half_sum(b, acc):
        buf = bufs[b]

        def row_body(r0, a):
            for u in range(RUNROLL):
                r = r0 * RUNROLL + u
                a = tuple(a[j] + buf[r, pl.ds(j * LANES, LANES)]
                          for j in range(NVREG))
            return a

        return lax.fori_loop(0, CHL // RUNROLL, row_body, acc)

    NG = BPW // SPG
    for b in range(NBUF):
        issue(b, b // NCHUNK, b % NCHUNK)

    def group_body(g, _):
        for p in range(SPG):
            si = g * SPG + p
            acc = tuple(jnp.zeros((LANES,), jnp.float32)
                        for _ in range(NVREG))
            for h in range(NCHUNK):
                b = p * NCHUNK + h
                wait(b)
                acc = half_sum(b, acc)

                @pl.when(g + 1 < NG)
                def _():
                    issue(b, (g + 1) * SPG + p, h)
            for j in range(NVREG):
                out_v[si, pl.ds(j * LANES, LANES)] = acc[j] * (1.0 / H)
        return 0

    lax.fori_loop(0, NG, group_body, 0)
    pltpu.sync_copy(out_v, out_hbm.at[pl.ds(base, BPW)])


@functools.partial(jax.jit, static_argnames=())
def kernel(x, table):
    mesh = plsc.VectorSubcoreMesh(core_axis_name="c", subcore_axis_name="s")
    f = pl.kernel(
        _sc_body,
        mesh=mesh,
        out_type=jax.ShapeDtypeStruct((B, D), jnp.float32),
        scratch_types=[
            pltpu.VMEM((BPW, NCHUNK, CHL), jnp.int32),
            *[pltpu.VMEM((CHL, D), jnp.float32) for _ in range(NBUF)],
            pltpu.VMEM((BPW, D), jnp.float32),
            *[pltpu.SemaphoreType.DMA for _ in range(NBUF)],
        ],
    )
    return f(x.astype(jnp.int32).reshape(B, NCHUNK, CHL), table)


# speedup vs baseline: 1.0040x; 1.0040x over previous
import functools

import jax
import jax.numpy as jnp
from jax import lax
from jax.experimental import pallas as pl
from jax.experimental.pallas import tpu as pltpu
from jax.experimental.pallas import tpu_sc as plsc

VOCAB = 100000
D = 128
B = 1024
H = 200

NUM_CORES = 2
NUM_SUBCORES = 16
NW = NUM_CORES * NUM_SUBCORES
BPW = B // NW
NCHUNK = 2
CHL = H // NCHUNK
LANES = 16
NVREG = D // LANES


NBUF = 8
RUNROLL = 4


def _sc_body(x_hbm, table_hbm, out_hbm, idx_v, *rest):
    c = lax.axis_index("c")
    s = lax.axis_index("s")
    wid = s * NUM_CORES + c
    base = wid * BPW

    pltpu.sync_copy(x_hbm.at[pl.ds(base, BPW)], idx_v)

    bufs = rest[:NBUF]
    out_v = rest[NBUF]
    sems = rest[NBUF + 1:]

    SPG = NBUF // NCHUNK

    def issue(b, si, h):
        pltpu.async_copy(table_hbm.at[idx_v.at[si, h]], bufs[b], sems[b])

    def wait(b):
        pltpu.make_async_copy(
            table_hbm.at[idx_v.at[0, 0]], bufs[b], sems[b]).wait()

    def half_sum(b, acc):
        buf = bufs[b]

        def row_body(r0, a):
            for u in range(RUNROLL):
                r = r0 * RUNROLL + u
                a = tuple(a[j] + buf[r, pl.ds(j * LANES, LANES)]
                          for j in range(NVREG))
            return a

        return lax.fori_loop(0, CHL // RUNROLL, row_body, acc)

    NG = BPW // SPG
    for b in range(NBUF):
        issue(b, b // NCHUNK, b % NCHUNK)

    def group_body(g, _):
        for p in range(SPG):
            si = g * SPG + p
            acc = tuple(jnp.zeros((LANES,), jnp.float32)
                        for _ in range(NVREG))
            for h in range(NCHUNK):
                b = p * NCHUNK + h
                wait(b)
                acc = half_sum(b, acc)

                @pl.when(g + 1 < NG)
                def _():
                    issue(b, (g + 1) * SPG + p, h)
            for j in range(NVREG):
                out_v[si, pl.ds(j * LANES, LANES)] = acc[j] * (1.0 / H)
        return 0

    lax.fori_loop(0, NG, group_body, 0)
    pltpu.sync_copy(out_v, out_hbm.at[pl.ds(base, BPW)])


@functools.partial(jax.jit, static_argnames=())
def kernel(x, table):
    mesh = plsc.VectorSubcoreMesh(core_axis_name="c", subcore_axis_name="s")
    f = pl.kernel(
        _sc_body,
        mesh=mesh,
        out_type=jax.ShapeDtypeStruct((B, D), jnp.float32),
        scratch_types=[
            pltpu.VMEM((BPW, NCHUNK, CHL), jnp.int32),
            *[pltpu.VMEM((CHL, D), jnp.float32) for _ in range(NBUF)],
            pltpu.VMEM((BPW, D), jnp.float32),
            *[pltpu.SemaphoreType.DMA for _ in range(NBUF)],
        ],
    )
    return f(x.astype(jnp.int32).reshape(B, NCHUNK, CHL), table)


# speedup vs baseline: 1.0042x; 1.0002x over previous
import functools

import jax
import jax.numpy as jnp
from jax import lax
from jax.experimental import pallas as pl
from jax.experimental.pallas import tpu as pltpu
from jax.experimental.pallas import tpu_sc as plsc

VOCAB = 100000
D = 128
B = 1024
H = 200

NUM_CORES = 2
NUM_SUBCORES = 16
NW = NUM_CORES * NUM_SUBCORES
BPW = B // NW
NCHUNK = 2
CHL = H // NCHUNK
LANES = 16
NVREG = D // LANES


NBUF = 8
RUNROLL = 4


def _sc_body(x_hbm, table_hbm, out_hbm, idx_v, *rest):
    c = lax.axis_index("c")
    s = lax.axis_index("s")
    wid = s * NUM_CORES + c
    base = wid * BPW

    pltpu.sync_copy(x_hbm.at[pl.ds(base, BPW)], idx_v)

    bufs = rest[:NBUF]
    out_v = rest[NBUF]
    sems = rest[NBUF + 1:]

    SPG = NBUF // NCHUNK

    def issue(b, si, h):
        pltpu.async_copy(table_hbm.at[idx_v.at[si, h]], bufs[b], sems[b])

    def wait(b):
        pltpu.make_async_copy(
            table_hbm.at[idx_v.at[0, 0]], bufs[b], sems[b]).wait()

    def half_sum(b, acc):
        buf = bufs[b]

        @plsc.parallel_loop(0, CHL // RUNROLL, carry=acc)
        def row_body(r0, a):
            for u in range(RUNROLL):
                r = r0 * RUNROLL + u
                a = tuple(a[j] + buf[r, pl.ds(j * LANES, LANES)]
                          for j in range(NVREG))
            return a

        return row_body

    NG = BPW // SPG
    for b in range(NBUF):
        issue(b, b // NCHUNK, b % NCHUNK)

    def group_body(g, _):
        for p in range(SPG):
            si = g * SPG + p
            acc = tuple(jnp.zeros((LANES,), jnp.float32)
                        for _ in range(NVREG))
            for h in range(NCHUNK):
                b = p * NCHUNK + h
                wait(b)
                acc = half_sum(b, acc)

                @pl.when(g + 1 < NG)
                def _():
                    issue(b, (g + 1) * SPG + p, h)
            for j in range(NVREG):
                out_v[si, pl.ds(j * LANES, LANES)] = acc[j] * (1.0 / H)
        return 0

    lax.fori_loop(0, NG, group_body, 0)
    pltpu.sync_copy(out_v, out_hbm.at[pl.ds(base, BPW)])


@functools.partial(jax.jit, static_argnames=())
def kernel(x, table):
    mesh = plsc.VectorSubcoreMesh(core_axis_name="c", subcore_axis_name="s")
    f = pl.kernel(
        _sc_body,
        mesh=mesh,
        out_type=jax.ShapeDtypeStruct((B, D), jnp.float32),
        scratch_types=[
            pltpu.VMEM((BPW, NCHUNK, CHL), jnp.int32),
            *[pltpu.VMEM((CHL, D), jnp.float32) for _ in range(NBUF)],
            pltpu.VMEM((BPW, D), jnp.float32),
            *[pltpu.SemaphoreType.DMA for _ in range(NBUF)],
        ],
    )
    return f(x.astype(jnp.int32).reshape(B, NCHUNK, CHL), table)
